# baseline (speedup 1.0000x reference)
import functools

import jax
import jax.numpy as jnp
from jax import lax
from jax.experimental import pallas as pl
from jax.experimental.pallas import tpu as pltpu
from jax.experimental.pallas import tpu_sc as plsc

EMB = 64
NC = 2
NS = 16
NW = NC * NS


@functools.partial(jax.jit, static_argnames=("n_rows",))
def _sc_embedding_gather(table, idx, n_rows):
    b_per_w = n_rows // NW
    chunk = 512
    n_chunks = b_per_w // chunk

    mesh = plsc.VectorSubcoreMesh(core_axis_name="c", subcore_axis_name="s")

    @functools.partial(
        pl.kernel,
        mesh=mesh,
        out_type=jax.ShapeDtypeStruct((n_rows, EMB), jnp.float32),
        scratch_types=[
            pltpu.VMEM((b_per_w,), jnp.int32),
            pltpu.VMEM((chunk, EMB), jnp.float32),
            pltpu.SemaphoreType.DMA,
        ],
        compiler_params=pltpu.CompilerParams(use_tc_tiling_on_sc=False),
    )
    def k(table_hbm, idx_hbm, out_hbm, idx_v, rows_v, sem):
        wid = lax.axis_index("s") * NC + lax.axis_index("c")
        base = wid * b_per_w
        pltpu.sync_copy(idx_hbm.at[pl.ds(base, b_per_w)], idx_v)

        def body(i, _):
            off = i * chunk
            pltpu.async_copy(
                table_hbm.at[idx_v.at[pl.ds(off, chunk)]], rows_v, sem
            ).wait()
            pltpu.sync_copy(rows_v, out_hbm.at[pl.ds(base + off, chunk)])
            return 0

        lax.fori_loop(0, n_chunks, body, 0)

    return k(table, idx)


def kernel(tok_ids, word_emb):
    flat = tok_ids.reshape(-1).astype(jnp.int32)
    out = _sc_embedding_gather(word_emb, flat, flat.shape[0])
    return out.reshape(tok_ids.shape + (EMB,))


# speedup vs baseline: 1.0264x; 1.0264x over previous
import functools

import jax
import jax.numpy as jnp
from jax import lax
from jax.experimental import pallas as pl
from jax.experimental.pallas import tpu as pltpu
from jax.experimental.pallas import tpu_sc as plsc

EMB = 64
NC = 2
NS = 16
NW = NC * NS
CHUNK = 320
NBUF = 4


@functools.partial(jax.jit, static_argnames=("n_rows",))
def _sc_embedding_gather(table, idx, n_rows):
    b_per_w = n_rows // NW
    n_chunks = b_per_w // CHUNK
    n_rounds = n_chunks // NBUF

    mesh = plsc.VectorSubcoreMesh(core_axis_name="c", subcore_axis_name="s")

    @functools.partial(
        pl.kernel,
        mesh=mesh,
        out_type=jax.ShapeDtypeStruct((n_rows, EMB), jnp.float32),
        scratch_types=[
            pltpu.VMEM((b_per_w,), jnp.int32),
            pltpu.VMEM((NBUF, CHUNK, EMB), jnp.float32),
            pltpu.SemaphoreType.DMA((NBUF,)),
            pltpu.SemaphoreType.DMA((NBUF,)),
        ],
        compiler_params=pltpu.CompilerParams(use_tc_tiling_on_sc=False),
    )
    def k(table_hbm, idx_hbm, out_hbm, idx_v, rows_v, gsem, wsem):
        wid = lax.axis_index("s") * NC + lax.axis_index("c")
        base = wid * b_per_w
        pltpu.sync_copy(idx_hbm.at[pl.ds(base, b_per_w)], idx_v)

        def start_gather(j, b):
            pltpu.async_copy(
                table_hbm.at[idx_v.at[pl.ds(j * CHUNK, CHUNK)]],
                rows_v.at[b],
                gsem.at[b],
            )

        def wait_gather(j, b):
            pltpu.make_async_copy(
                table_hbm.at[idx_v.at[pl.ds(j * CHUNK, CHUNK)]],
                rows_v.at[b],
                gsem.at[b],
            ).wait()

        def start_wb(j, b):
            pltpu.async_copy(
                rows_v.at[b],
                out_hbm.at[pl.ds(base + j * CHUNK, CHUNK)],
                wsem.at[b],
            )

        def wait_wb(j, b):
            pltpu.make_async_copy(
                rows_v.at[b],
                out_hbm.at[pl.ds(base + j * CHUNK, CHUNK)],
                wsem.at[b],
            ).wait()

        for b in range(NBUF):
            start_gather(b, b)

        def round_body(r, _):
            j0 = r * NBUF
            for b in range(NBUF):
                j = j0 + b
                wait_gather(j, b)
                start_wb(j, b)
                wait_wb(j, b)
                start_gather(j + NBUF, b)
            return 0

        lax.fori_loop(0, n_rounds - 1, round_body, 0)

        j0 = (n_rounds - 1) * NBUF
        for b in range(NBUF):
            j = j0 + b
            wait_gather(j, b)
            start_wb(j, b)
        for b in range(NBUF):
            wait_wb(j0 + b, b)

    return k(table, idx)


def kernel(tok_ids, word_emb):
    flat = tok_ids.reshape(-1).astype(jnp.int32)
    out = _sc_embedding_gather(word_emb, flat, flat.shape[0])
    return out.reshape(tok_ids.shape + (EMB,))


# speedup vs baseline: 1.5013x; 1.4627x over previous
import functools

import jax
import jax.numpy as jnp
from jax import lax
from jax.experimental import pallas as pl
from jax.experimental.pallas import tpu as pltpu
from jax.experimental.pallas import tpu_sc as plsc

VOC = 1000000
EMB = 64
BATCH = 4096
SEQ = 200
NC = 2
NS = 16
NW = NC * NS


def _sc_lookup(table, tok_t):
    mesh = plsc.VectorSubcoreMesh(core_axis_name="c", subcore_axis_name="s")

    @functools.partial(
        pl.kernel,
        mesh=mesh,
        out_type=jax.ShapeDtypeStruct((SEQ, 8, NW, 8, 128), jnp.float32),
        scratch_types=[
            pltpu.VMEM((SEQ, 128), jnp.int32),
            pltpu.VMEM((2, 128, EMB), jnp.float32),
            pltpu.VMEM((2, 8, 8, 128), jnp.float32),
            pltpu.SemaphoreType.DMA((2,)),
            pltpu.SemaphoreType.DMA((2,)),
        ],
        compiler_params=pltpu.CompilerParams(use_tc_tiling_on_sc=False),
    )
    def k(tab_hbm, tok_hbm, out_hbm, tok_all, rows, stripe, gsem, wsem):
        wid = lax.axis_index("s") * NC + lax.axis_index("c")
        iota = lax.iota(jnp.int32, 16)

        keep = [(iota & (1 << t)) == 0 for t in range(4)]
        down = [(iota - (1 << t)) & 15 for t in range(4)]
        up = [(iota + (1 << t)) & 15 for t in range(4)]

        pltpu.sync_copy(tok_hbm.at[:, pl.ds(wid * 128, 128)], tok_all)

        def start_gather(s, b):
            pltpu.async_copy(tab_hbm.at[tok_all.at[s]], rows.at[b], gsem.at[b])

        def wait_gather(s, b):
            pltpu.make_async_copy(
                tab_hbm.at[tok_all.at[s]], rows.at[b], gsem.at[b]
            ).wait()

        def start_write(s, b):
            pltpu.async_copy(
                stripe.at[b], out_hbm.at[s, :, wid, :, :], wsem.at[b]
            )

        def wait_write(s, b):
            pltpu.make_async_copy(
                stripe.at[b], out_hbm.at[s, :, wid, :, :], wsem.at[b]
            ).wait()

        def transpose16(vecs):
            for t in range(4):
                m = 1 << t
                out = list(vecs)
                for p in range(16):
                    if p & m:
                        continue
                    q = p | m
                    a, c = vecs[p], vecs[q]
                    out[p] = jnp.where(keep[t], a, jnp.take(c, down[t]))
                    out[q] = jnp.where(keep[t], jnp.take(a, up[t]), c)
                vecs = out
            return vecs

        def assemble(b):
            for e0 in range(0, EMB, 16):
                eh0, el0 = e0 // 8, e0 % 8

                def kblock(kb, _):
                    k0 = kb * 16
                    vecs = [rows[b, k0 + i, pl.ds(e0, 16)] for i in range(16)]
                    tv = transpose16(vecs)
                    for j in range(16):
                        eh, el = eh0 + (el0 + j) // 8, (el0 + j) % 8
                        stripe[b, eh, el, pl.ds(k0, 16)] = tv[j]
                    return 0

                lax.fori_loop(0, 8, kblock, 0)

        n = SEQ
        start_gather(0, 0)
        start_gather(1, 1)
        for b in range(2):
            wait_gather(b, b)
            assemble(b)
            start_write(b, b)
            start_gather(b + 2, b)

        def round_body(r, _):
            for b in range(2):
                s = 2 * r + b
                wait_gather(s, b)
                wait_write(s - 2, b)
                assemble(b)
                start_write(s, b)
                start_gather(s + 2, b)
            return 0

        lax.fori_loop(1, n // 2 - 1, round_body, 0)

        for b in range(2):
            s = n - 2 + b
            wait_gather(s, b)
            wait_write(s - 2, b)
            assemble(b)
            start_write(s, b)
        for b in range(2):
            wait_write(n - 2 + b, b)

    return k(table, tok_t)


def kernel(tok_ids, word_emb):
    tok_t = tok_ids.T.astype(jnp.int32)
    out5 = _sc_lookup(word_emb, tok_t)
    return out5.transpose(2, 4, 0, 1, 3).reshape(BATCH, SEQ, EMB)


# speedup vs baseline: 1.5320x; 1.0204x over previous
import functools

import jax
import jax.numpy as jnp
from jax import lax
from jax.experimental import pallas as pl
from jax.experimental.pallas import tpu as pltpu
from jax.experimental.pallas import tpu_sc as plsc

VOC = 1000000
EMB = 64
BATCH = 4096
SEQ = 200
NC = 2
NS = 16
NW = NC * NS


def _sc_lookup(table, tok_t):
    mesh = plsc.VectorSubcoreMesh(core_axis_name="c", subcore_axis_name="s")

    @functools.partial(
        pl.kernel,
        mesh=mesh,
        out_type=jax.ShapeDtypeStruct((SEQ, 8, NW, 8, 128), jnp.float32),
        scratch_types=[
            pltpu.VMEM((SEQ, 128), jnp.int32),
            pltpu.VMEM((2, 128, 128), jnp.float32),
            pltpu.VMEM((2, 8, 8, 128), jnp.float32),
            pltpu.SemaphoreType.DMA((2,)),
            pltpu.SemaphoreType.DMA((2,)),
        ],
        compiler_params=pltpu.CompilerParams(use_tc_tiling_on_sc=False),
    )
    def k(tab_hbm, tok_hbm, out_hbm, tok_all, rows, stripe, gsem, wsem):
        wid = lax.axis_index("s") * NC + lax.axis_index("c")
        iota = lax.iota(jnp.int32, 16)

        keep = [(iota & (1 << t)) == 0 for t in range(4)]
        down = [(iota - (1 << t)) & 15 for t in range(4)]
        up = [(iota + (1 << t)) & 15 for t in range(4)]

        pltpu.sync_copy(tok_hbm.at[:, pl.ds(wid * 128, 128)], tok_all)

        def start_gather(s, b):
            pltpu.async_copy(tab_hbm.at[tok_all.at[s]], rows.at[b], gsem.at[b])

        def wait_gather(s, b):
            pltpu.make_async_copy(
                tab_hbm.at[tok_all.at[s]], rows.at[b], gsem.at[b]
            ).wait()

        def start_write(s, b):
            pltpu.async_copy(
                stripe.at[b], out_hbm.at[s, :, wid, :, :], wsem.at[b]
            )

        def wait_write(s, b):
            pltpu.make_async_copy(
                stripe.at[b], out_hbm.at[s, :, wid, :, :], wsem.at[b]
            ).wait()

        def transpose16(vecs):
            for t in range(4):
                m = 1 << t
                out = list(vecs)
                for p in range(16):
                    if p & m:
                        continue
                    q = p | m
                    a, c = vecs[p], vecs[q]
                    out[p] = jnp.where(keep[t], a, jnp.take(c, down[t]))
                    out[q] = jnp.where(keep[t], jnp.take(a, up[t]), c)
                vecs = out
            return vecs

        def assemble(b):
            for e0 in range(0, EMB, 16):
                eh0, el0 = e0 // 8, e0 % 8

                def kblock(kb, _):
                    k0 = kb * 16
                    vecs = [rows[b, k0 + i, pl.ds(e0, 16)] for i in range(16)]
                    tv = transpose16(vecs)
                    for j in range(16):
                        eh, el = eh0 + (el0 + j) // 8, (el0 + j) % 8
                        stripe[b, eh, el, pl.ds(k0, 16)] = tv[j]
                    return 0

                lax.fori_loop(0, 8, kblock, 0)

        n = SEQ
        start_gather(0, 0)
        start_gather(1, 1)
        for b in range(2):
            wait_gather(b, b)
            assemble(b)
            start_write(b, b)
            start_gather(b + 2, b)

        def round_body(r, _):
            for b in range(2):
                s = 2 * r + b
                wait_gather(s, b)
                wait_write(s - 2, b)
                assemble(b)
                start_write(s, b)
                start_gather(s + 2, b)
            return 0

        lax.fori_loop(1, n // 2 - 1, round_body, 0)

        for b in range(2):
            s = n - 2 + b
            wait_gather(s, b)
            wait_write(s - 2, b)
            assemble(b)
            start_write(s, b)
        for b in range(2):
            wait_write(n - 2 + b, b)

    return k(table, tok_t)


def kernel(tok_ids, word_emb):
    tok_t = tok_ids.T.astype(jnp.int32)
    wpad = jnp.pad(word_emb, ((0, 0), (0, 128 - EMB)))
    out5 = _sc_lookup(wpad, tok_t)
    return out5.transpose(2, 4, 0, 1, 3).reshape(BATCH, SEQ, EMB)


# speedup vs baseline: 1.6178x; 1.0560x over previous
import functools

import jax
import jax.numpy as jnp
from jax import lax
from jax.experimental import pallas as pl
from jax.experimental.pallas import tpu as pltpu
from jax.experimental.pallas import tpu_sc as plsc

VOC = 1000000
EMB = 64
BATCH = 4096
SEQ = 200
NC = 2
NS = 16
NW = NC * NS


def _sc_lookup(table, tok_t):
    mesh = plsc.VectorSubcoreMesh(core_axis_name="c", subcore_axis_name="s")

    @functools.partial(
        pl.kernel,
        mesh=mesh,
        out_type=jax.ShapeDtypeStruct((SEQ, 8, NW, 8, 128), jnp.float32),
        scratch_types=[
            pltpu.VMEM((SEQ, 128), jnp.int32),
            pltpu.VMEM((2, 128), jnp.int32),
            pltpu.VMEM((2, 128, 64), jnp.float32),
            pltpu.VMEM((2, 8, 8, 128), jnp.float32),
            pltpu.SemaphoreType.DMA((2,)),
            pltpu.SemaphoreType.DMA((2,)),
        ],
        compiler_params=pltpu.CompilerParams(use_tc_tiling_on_sc=False),
    )
    def k(tab_hbm, tok_hbm, out_hbm, tok_all, idx2, rows, stripe, gsem, wsem):
        wid = lax.axis_index("s") * NC + lax.axis_index("c")
        iota = lax.iota(jnp.int32, 16)

        keep = [(iota & (1 << t)) == 0 for t in range(4)]
        down = [(iota - (1 << t)) & 15 for t in range(4)]
        up = [(iota + (1 << t)) & 15 for t in range(4)]

        pltpu.sync_copy(tok_hbm.at[:, pl.ds(wid * 128, 128)], tok_all)

        def prep(s, b):
            for k0 in range(0, 128, 16):
                idx2[b, pl.ds(k0, 16)] = tok_all[s, pl.ds(k0, 16)] * 2

        def start_gather(b):
            pltpu.async_copy(tab_hbm.at[idx2.at[b]], rows.at[b], gsem.at[b])

        def wait_gather(b):
            pltpu.make_async_copy(
                tab_hbm.at[idx2.at[b]], rows.at[b], gsem.at[b]
            ).wait()

        def start_write(s, b):
            pltpu.async_copy(
                stripe.at[b], out_hbm.at[s, :, wid, :, :], wsem.at[b]
            )

        def wait_write(s, b):
            pltpu.make_async_copy(
                stripe.at[b], out_hbm.at[s, :, wid, :, :], wsem.at[b]
            ).wait()

        def transpose16(vecs):
            for t in range(4):
                m = 1 << t
                out = list(vecs)
                for p in range(16):
                    if p & m:
                        continue
                    q = p | m
                    a, c = vecs[p], vecs[q]
                    out[p] = jnp.where(keep[t], a, jnp.take(c, down[t]))
                    out[q] = jnp.where(keep[t], jnp.take(a, up[t]), c)
                vecs = out
            return vecs

        def assemble(b):
            for e0 in range(0, EMB, 16):
                eh0, el0 = e0 // 8, e0 % 8

                def kblock(kb, _):
                    k0 = kb * 16
                    vecs = [rows[b, k0 + i, pl.ds(e0, 16)] for i in range(16)]
                    tv = transpose16(vecs)
                    for j in range(16):
                        eh, el = eh0 + (el0 + j) // 8, (el0 + j) % 8
                        stripe[b, eh, el, pl.ds(k0, 16)] = tv[j]
                    return 0

                lax.fori_loop(0, 8, kblock, 0)

        n = SEQ
        for b in range(2):
            prep(b, b)
            start_gather(b)
        for b in range(2):
            wait_gather(b)
            assemble(b)
            start_write(b, b)
            prep(b + 2, b)
            start_gather(b)

        def round_body(r, _):
            for b in range(2):
                s = 2 * r + b
                wait_gather(b)
                wait_write(s - 2, b)
                assemble(b)
                start_write(s, b)
                prep(s + 2, b)
                start_gather(b)
            return 0

        lax.fori_loop(1, n // 2 - 1, round_body, 0)

        for b in range(2):
            s = n - 2 + b
            wait_gather(b)
            wait_write(s - 2, b)
            assemble(b)
            start_write(s, b)
        for b in range(2):
            wait_write(n - 2 + b, b)

    return k(table, tok_t)


def kernel(tok_ids, word_emb):
    tok_t = tok_ids.T.astype(jnp.int32)
    wpad = jnp.pad(word_emb, ((0, 0), (0, 128 - EMB)))
    wview = wpad.reshape(2 * VOC, EMB)
    out5 = _sc_lookup(wview, tok_t)
    return out5.transpose(2, 4, 0, 1, 3).reshape(BATCH, SEQ, EMB)
